# baseline (speedup 1.0000x reference)
import functools

import jax
import jax.numpy as jnp
from jax import lax
from jax.experimental import pallas as pl
from jax.experimental.pallas import tpu as pltpu
from jax.experimental.pallas import tpu_sc as plsc

N = 10000
E = 320000
C = 128
K = 6
EPS = 1e-5

NC = 2
NS = 16
NW = NC * NS
EPW = E // NW
CH = 40
NCHUNK = EPW // CH
NBUF = 5
NGRP = NCHUNK // NBUF
NPAD = 10240
ROWS_PER = NPAD // NS

_mesh = plsc.VectorSubcoreMesh(core_axis_name="c", subcore_axis_name="s")


@functools.partial(
    pl.kernel,
    out_type=jax.ShapeDtypeStruct((NC, NPAD, C), jnp.float32),
    mesh=_mesh,
    scratch_types=(
        [pltpu.VMEM((2, NBUF, CH), jnp.int32),
         pltpu.VMEM((2, NBUF, CH), jnp.int32)]
        + [pltpu.VMEM((CH, C), jnp.float32) for _ in range(NBUF)]
        + [pltpu.SemaphoreType.DMA for _ in range(NBUF)]
        + [pltpu.SemaphoreType.DMA, pltpu.VMEM_SHARED((NPAD, C), jnp.float32)]
    ),
)
def _sc_segsum(g_hbm, col_hbm, row_hbm, zeros_hbm, out_hbm, colg, rowg,
               *rest):
    bufs = rest[:NBUF]
    gsems = rest[NBUF:2 * NBUF]
    isem = rest[2 * NBUF]
    accum = rest[2 * NBUF + 1]
    cid = lax.axis_index("c")
    sid = lax.axis_index("s")
    wid = sid * NC + cid
    r0 = sid * ROWS_PER
    pltpu.sync_copy(zeros_hbm.at[pl.ds(r0, ROWS_PER)],
                    accum.at[pl.ds(r0, ROWS_PER)])
    pltpu.sync_copy(col_hbm.at[wid, 0], colg.at[0])
    pltpu.sync_copy(row_hbm.at[wid, 0], rowg.at[0])
    pltpu.async_copy(col_hbm.at[wid, 1], colg.at[1], isem)
    pltpu.async_copy(row_hbm.at[wid, 1], rowg.at[1], isem)
    plsc.subcore_barrier()

    for b in range(NBUF):
        pltpu.async_copy(g_hbm.at[colg.at[0, b]], bufs[b], gsems[b])

    def body(g, carry):
        p = lax.rem(g, 2)
        pn = lax.rem(g + 1, 2)

        @pl.when(g + 1 < NGRP)
        def _():
            pltpu.make_async_copy(col_hbm.at[wid, 0], colg.at[pn],
                                  isem).wait()
            pltpu.make_async_copy(row_hbm.at[wid, 0], rowg.at[pn],
                                  isem).wait()

        for b in range(NBUF):
            pltpu.make_async_copy(g_hbm.at[colg.at[p, b]], bufs[b],
                                  gsems[b]).wait()
            pltpu.sync_copy(bufs[b], accum.at[rowg.at[p, b]], add=True)

            @pl.when(g + 1 < NGRP)
            def _():
                pltpu.async_copy(g_hbm.at[colg.at[pn, b]], bufs[b],
                                 gsems[b])

        @pl.when(g + 2 < NGRP)
        def _():
            pltpu.async_copy(col_hbm.at[wid, g + 2], colg.at[p], isem)
            pltpu.async_copy(row_hbm.at[wid, g + 2], rowg.at[p], isem)

        return carry

    lax.fori_loop(0, NGRP, body, 0)
    plsc.subcore_barrier()
    pltpu.sync_copy(accum.at[pl.ds(r0, ROWS_PER)],
                    out_hbm.at[cid, pl.ds(r0, ROWS_PER)])


_B = 1000
_GRID = N // _B


def _tc_prep_body(x_ref, d16_ref, w0_ref, dinvb_ref, g0_ref, acc_ref):
    deg = d16_ref[0, :, 0:1] + d16_ref[1, :, 0:1]
    dinv = jnp.where(deg > 0.0, lax.rsqrt(jnp.maximum(deg, 1e-12)), 0.0)
    dinvb = jnp.broadcast_to(dinv, (_B, C))
    x = x_ref[...]
    dinvb_ref[...] = dinvb
    g0_ref[...] = dinvb * x
    acc_ref[...] = jnp.dot(x, w0_ref[...], preferred_element_type=jnp.float32)


def _tc_combine_body(first, yp_ref, dinvb_ref, txm2_ref, acc_ref, wk_ref,
                     tx_ref, g_ref, accout_ref):
    y = yp_ref[0] + yp_ref[1]
    dinvb = dinvb_ref[...]
    if first:
        tx = -(dinvb * y)
    else:
        tx = -2.0 * (dinvb * y) - txm2_ref[...]
    tx_ref[...] = tx
    g_ref[...] = dinvb * tx
    accout_ref[...] = acc_ref[...] + jnp.dot(
        tx, wk_ref[...], preferred_element_type=jnp.float32)


def _tc_final_body(acc_ref, b_ref, o_ref):
    h = acc_ref[...] + b_ref[...]
    m = jnp.mean(h, axis=1, keepdims=True)
    cen = h - m
    v = jnp.mean(cen * cen, axis=1, keepdims=True)
    o_ref[...] = jnp.maximum(cen * lax.rsqrt(v + EPS), 0.0)


_row_spec = pl.BlockSpec((_B, C), lambda i: (i, 0))
_w_spec = pl.BlockSpec((C, C), lambda i: (0, 0))

_tc_prep = pl.pallas_call(
    _tc_prep_body,
    grid=(_GRID,),
    in_specs=[_row_spec,
              pl.BlockSpec((NC, _B, C), lambda i: (0, i, 0)),
              _w_spec],
    out_specs=[_row_spec, _row_spec, _row_spec],
    out_shape=[jax.ShapeDtypeStruct((N, C), jnp.float32)] * 3,
)

_yp_spec = pl.BlockSpec((NC, _B, C), lambda i: (0, i, 0))

_tc_combine_first = pl.pallas_call(
    functools.partial(_tc_combine_body, True),
    grid=(_GRID,),
    in_specs=[_yp_spec, _row_spec, _row_spec, _row_spec, _w_spec],
    out_specs=[_row_spec, _row_spec, _row_spec],
    out_shape=[jax.ShapeDtypeStruct((N, C), jnp.float32)] * 3,
)

_tc_combine_rest = pl.pallas_call(
    functools.partial(_tc_combine_body, False),
    grid=(_GRID,),
    in_specs=[_yp_spec, _row_spec, _row_spec, _row_spec, _w_spec],
    out_specs=[_row_spec, _row_spec, _row_spec],
    out_shape=[jax.ShapeDtypeStruct((N, C), jnp.float32)] * 3,
)

_tc_final = pl.pallas_call(
    _tc_final_body,
    grid=(_GRID,),
    in_specs=[_row_spec, pl.BlockSpec((1, C), lambda i: (0, 0))],
    out_specs=_row_spec,
    out_shape=jax.ShapeDtypeStruct((N, C), jnp.float32),
)


def kernel(x, adj_indices, W, b):
    row = adj_indices[0].reshape(NW, NGRP, NBUF, CH)
    col = adj_indices[1].reshape(NW, NGRP, NBUF, CH)
    zeros_nc = jnp.zeros((NPAD, C), jnp.float32)
    ones_n = jnp.ones((N, C), jnp.float32)
    zero_idx = jnp.zeros((NW, NGRP, NBUF, CH), jnp.int32)

    d16 = _sc_segsum(ones_n, zero_idx, row, zeros_nc)
    dinvb, g, acc = _tc_prep(x, d16, W[0])

    tx_pp = x
    tx_p = x
    for k in range(1, K):
        yp = _sc_segsum(g, col, row, zeros_nc)
        if k == 1:
            tx, g, acc = _tc_combine_first(yp, dinvb, tx_p, acc, W[k])
        else:
            tx, g, acc = _tc_combine_rest(yp, dinvb, tx_pp, acc, W[k])
        tx_pp, tx_p = tx_p, tx

    return _tc_final(acc, b.reshape(1, C))


# speedup vs baseline: 15.0456x; 15.0456x over previous
import functools

import jax
import jax.numpy as jnp
from jax import lax
from jax.experimental import pallas as pl
from jax.experimental.pallas import tpu as pltpu
from jax.experimental.pallas import tpu_sc as plsc

N = 10000
E = 320000
C = 128
K = 6
EPS = 1e-5

NC = 2
NS = 16
NW = NC * NS
EPW = E // NW
CH = 40
NCHUNK = EPW // CH
NBUF = 5
NGRP = NCHUNK // NBUF
NPAD = 10240
ROWS_PER = NPAD // NS

_mesh = plsc.VectorSubcoreMesh(core_axis_name="c", subcore_axis_name="s")


@functools.partial(
    pl.kernel,
    out_type=jax.ShapeDtypeStruct((NC, NPAD, C), jnp.float32),
    mesh=_mesh,
    scratch_types=(
        [pltpu.VMEM((2, NBUF, CH), jnp.int32),
         pltpu.VMEM((2, NBUF, CH), jnp.int32)]
        + [pltpu.VMEM((CH, C), jnp.float32) for _ in range(NBUF)]
        + [pltpu.SemaphoreType.DMA for _ in range(NBUF)]
        + [pltpu.SemaphoreType.DMA, pltpu.VMEM_SHARED((NPAD, C), jnp.float32)]
    ),
)
def _sc_segsum(g_hbm, col_hbm, row_hbm, zeros_hbm, out_hbm, colg, rowg,
               *rest):
    bufs = rest[:NBUF]
    gsems = rest[NBUF:2 * NBUF]
    isem = rest[2 * NBUF]
    accum = rest[2 * NBUF + 1]
    cid = lax.axis_index("c")
    sid = lax.axis_index("s")
    wid = sid * NC + cid
    r0 = sid * ROWS_PER
    pltpu.sync_copy(zeros_hbm.at[pl.ds(r0, ROWS_PER)],
                    accum.at[pl.ds(r0, ROWS_PER)])
    pltpu.sync_copy(col_hbm.at[wid, 0], colg.at[0])
    pltpu.sync_copy(row_hbm.at[wid, 0], rowg.at[0])
    pltpu.async_copy(col_hbm.at[wid, 1], colg.at[1], isem)
    pltpu.async_copy(row_hbm.at[wid, 1], rowg.at[1], isem)
    plsc.subcore_barrier()

    for b in range(NBUF):
        pltpu.async_copy(g_hbm.at[colg.at[0, b]], bufs[b], gsems[b])

    def body(g, carry):
        p = lax.rem(g, 2)
        pn = lax.rem(g + 1, 2)

        @pl.when(g + 1 < NGRP)
        def _():
            pltpu.make_async_copy(col_hbm.at[wid, 0], colg.at[pn],
                                  isem).wait()
            pltpu.make_async_copy(row_hbm.at[wid, 0], rowg.at[pn],
                                  isem).wait()

        for b in range(NBUF):
            pltpu.make_async_copy(g_hbm.at[colg.at[p, b]], bufs[b],
                                  gsems[b]).wait()
            pltpu.sync_copy(bufs[b], accum.at[rowg.at[p, b]], add=True)

            @pl.when(g + 1 < NGRP)
            def _():
                pltpu.async_copy(g_hbm.at[colg.at[pn, b]], bufs[b],
                                 gsems[b])

        @pl.when(g + 2 < NGRP)
        def _():
            pltpu.async_copy(col_hbm.at[wid, g + 2], colg.at[p], isem)
            pltpu.async_copy(row_hbm.at[wid, g + 2], rowg.at[p], isem)

        return carry

    lax.fori_loop(0, NGRP, body, 0)
    plsc.subcore_barrier()
    pltpu.sync_copy(accum.at[pl.ds(r0, ROWS_PER)],
                    out_hbm.at[cid, pl.ds(r0, ROWS_PER)])


_B = 1000
_GRID = N // _B


def _tc_prep_body(x_ref, d16_ref, w0_ref, dinvb_ref, g0_ref, acc_ref):
    deg = d16_ref[0, :, 0:1] + d16_ref[1, :, 0:1]
    dinv = jnp.where(deg > 0.0, lax.rsqrt(jnp.maximum(deg, 1e-12)), 0.0)
    dinvb = jnp.broadcast_to(dinv, (_B, C))
    x = x_ref[...]
    dinvb_ref[...] = dinvb
    g0_ref[...] = dinvb * x
    acc_ref[...] = jnp.dot(x, w0_ref[...], preferred_element_type=jnp.float32)


def _tc_combine_body(first, yp_ref, dinvb_ref, txm2_ref, acc_ref, wk_ref,
                     tx_ref, g_ref, accout_ref):
    y = yp_ref[0] + yp_ref[1]
    dinvb = dinvb_ref[...]
    if first:
        tx = -(dinvb * y)
    else:
        tx = -2.0 * (dinvb * y) - txm2_ref[...]
    tx_ref[...] = tx
    g_ref[...] = dinvb * tx
    accout_ref[...] = acc_ref[...] + jnp.dot(
        tx, wk_ref[...], preferred_element_type=jnp.float32)


def _tc_final_body(acc_ref, b_ref, o_ref):
    h = acc_ref[...] + b_ref[...]
    m = jnp.mean(h, axis=1, keepdims=True)
    cen = h - m
    v = jnp.mean(cen * cen, axis=1, keepdims=True)
    o_ref[...] = jnp.maximum(cen * lax.rsqrt(v + EPS), 0.0)


_row_spec = pl.BlockSpec((_B, C), lambda i: (i, 0))
_w_spec = pl.BlockSpec((C, C), lambda i: (0, 0))

_tc_prep = pl.pallas_call(
    _tc_prep_body,
    grid=(_GRID,),
    in_specs=[_row_spec,
              pl.BlockSpec((NC, _B, C), lambda i: (0, i, 0)),
              _w_spec],
    out_specs=[_row_spec, _row_spec, _row_spec],
    out_shape=[jax.ShapeDtypeStruct((N, C), jnp.float32)] * 3,
)

_yp_spec = pl.BlockSpec((NC, _B, C), lambda i: (0, i, 0))

_tc_combine_first = pl.pallas_call(
    functools.partial(_tc_combine_body, True),
    grid=(_GRID,),
    in_specs=[_yp_spec, _row_spec, _row_spec, _row_spec, _w_spec],
    out_specs=[_row_spec, _row_spec, _row_spec],
    out_shape=[jax.ShapeDtypeStruct((N, C), jnp.float32)] * 3,
)

_tc_combine_rest = pl.pallas_call(
    functools.partial(_tc_combine_body, False),
    grid=(_GRID,),
    in_specs=[_yp_spec, _row_spec, _row_spec, _row_spec, _w_spec],
    out_specs=[_row_spec, _row_spec, _row_spec],
    out_shape=[jax.ShapeDtypeStruct((N, C), jnp.float32)] * 3,
)

_tc_final = pl.pallas_call(
    _tc_final_body,
    grid=(_GRID,),
    in_specs=[_row_spec, pl.BlockSpec((1, C), lambda i: (0, 0))],
    out_specs=_row_spec,
    out_shape=jax.ShapeDtypeStruct((N, C), jnp.float32),
)


def kernel(x, adj_indices, W, b):
    row = adj_indices[0].reshape(NW, NGRP, NBUF, CH)
    col = adj_indices[1].reshape(NW, NGRP, NBUF, CH)
    zeros_nc = jnp.zeros((NPAD, C), jnp.float32)
    ones_n = jnp.ones((N, C), jnp.float32)

    d16 = _sc_segsum(ones_n, col, row, zeros_nc)
    dinvb, g, acc = _tc_prep(x, d16, W[0])

    tx_pp = x
    tx_p = x
    for k in range(1, K):
        yp = _sc_segsum(g, col, row, zeros_nc)
        if k == 1:
            tx, g, acc = _tc_combine_first(yp, dinvb, tx_p, acc, W[k])
        else:
            tx, g, acc = _tc_combine_rest(yp, dinvb, tx_pp, acc, W[k])
        tx_pp, tx_p = tx_p, tx

    return _tc_final(acc, b.reshape(1, C))
